# baseline (speedup 1.0000x reference)
import functools

import jax
import jax.numpy as jnp
from jax import lax
from jax.experimental import pallas as pl
from jax.experimental.pallas import tpu as pltpu
from jax.experimental.pallas import tpu_sc as plsc

_N_ROWS = 16384
_N_COLS = 26
_B = _N_ROWS * _N_COLS
_NC = 2
_NS = 16
_NW = _NC * _NS
_PER_W = _B // _NW
_NCH = 4
_CHW = _PER_W // _NCH
_ROWS_W = _N_ROWS // _NW

_mesh = plsc.VectorSubcoreMesh(core_axis_name="c", subcore_axis_name="s")


@functools.partial(
    pl.kernel,
    mesh=_mesh,
    out_type=jax.ShapeDtypeStruct((_N_ROWS, _N_COLS), jnp.float32),
    scratch_types=[
        pltpu.VMEM((_PER_W,), jnp.int32),
        pltpu.VMEM((_PER_W,), jnp.float32),
        pltpu.VMEM((_ROWS_W, _N_COLS), jnp.float32),
        pltpu.SemaphoreType.DMA,
        pltpu.SemaphoreType.DMA,
        pltpu.SemaphoreType.DMA,
        pltpu.SemaphoreType.DMA,
        pltpu.SemaphoreType.DMA,
    ],
)
def _gather_kernel(idx_hbm, table_hbm, out_hbm, idx_v, vals_v,
                   vals2_v, sem_g0, sem_g1, sem_g2, sem_g3, sem_o):
    wid = lax.axis_index("s") * _NC + lax.axis_index("c")
    base = wid * _PER_W
    row0 = wid * _ROWS_W
    pltpu.sync_copy(idx_hbm.at[pl.ds(base, _PER_W)], idx_v)

    sems = (sem_g0, sem_g1, sem_g2, sem_g3)
    gathers = []
    for k in range(_NCH):
        o = k * _CHW
        gathers.append(pltpu.async_copy(
            table_hbm.at[idx_v.at[pl.ds(o, _CHW)]],
            vals_v.at[pl.ds(o, _CHW)], sems[k]))
    for k in range(_NCH):
        gathers[k].wait()

    def _pack_body(r, _):
        f = r * _N_COLS
        vals2_v[r, pl.ds(0, 16)] = vals_v[pl.ds(f, 16)]
        vals2_v[r, pl.ds(10, 16)] = vals_v[pl.ds(f + 10, 16)]
        return _

    lax.fori_loop(0, _ROWS_W, _pack_body, 0, unroll=8)

    pltpu.sync_copy(vals2_v, out_hbm.at[pl.ds(row0, _ROWS_W), :])


def kernel(states, potential_weights):
    return _gather_kernel(states.reshape(-1).astype(jnp.int32),
                          potential_weights)
